# baseline (speedup 1.0000x reference)
import numpy as np
import jax
import jax.numpy as jnp
from jax.experimental import pallas as pl
from jax.experimental.pallas import tpu as pltpu


K = 5
H1IN, W1IN = 28, 28
P1H, P1W = 12, 12
C1 = 10
H2IN, W2IN = 12, 12
P2H, P2W = 4, 4
C2 = 20
C2P = 24

XPAD = 896
K1 = 384
N1 = 1536
H1P = 1536
K2 = 1024
N2 = 1024
G2 = 384
FPAD = 384

S1 = (0, 1, 28, 29)
S2 = (0, 10, 120, 130)


def _sel1_np():
    j = np.arange(K1)[:, None, None]
    q = np.arange(4 * 3 * P1W)[None, :, None]
    off, ql = q // (3 * P1W), q % (3 * P1W)
    phl, pw = ql // P1W, ql % P1W
    s1 = np.asarray(S1)[off]
    t = np.arange(32)[None, None, :]
    ky, kx = t // K, t % K
    sel = (t < K * K) & (j - s1 == (2 * phl + ky) * W1IN + 2 * pw + kx)
    return sel.astype(np.float32)


def _sel2_np():
    jr = np.arange(K2)[:, None, None]
    q = np.arange(4 * 2 * P2W)[None, :, None]
    off, ql = q // (2 * P2W), q % (2 * P2W)
    phl2, pw2 = ql // P2W, ql % P2W
    s2 = np.asarray(S2)[off]
    ct = np.arange(256)[None, None, :]
    ci, t = ct // (K * K), ct % (K * K)
    ky, kx = t // K, t % K
    sel = ((ct < C1 * K * K)
           & (jr - s2 == ((2 * phl2 + ky) * W2IN + 2 * pw2 + kx) * C1 + ci))
    return sel.astype(np.float32)


def _perm3_np():
    s = np.arange(G2)
    p2, c_out = s // C2P, s % C2P
    i = np.arange(C2 * P2H * P2W)[None, :]
    sel = (c_out[:, None] < C2) & (i == c_out[:, None] * (P2H * P2W) + p2[:, None])
    return sel.astype(np.float32)


_SEL1 = _sel1_np()
_SEL2 = _sel2_np()
_PERM3 = _perm3_np()


def _net_kernel(x_ref, w1_ref, b1_ref, w2_ref, b2_ref, w3_ref, b3_ref, o_ref):
    xb = jnp.pad(x_ref[...].astype(jnp.bfloat16),
                 ((0, 0), (0, XPAD - H1IN * W1IN)))
    w1 = w1_ref[...]
    parts = []
    for g in range(4):
        z = jnp.dot(xb[:, 168 * g:168 * g + K1], w1,
                    preferred_element_type=jnp.float32)
        m = jnp.maximum(jnp.maximum(z[:, 0:384], z[:, 384:768]),
                        jnp.maximum(z[:, 768:1152], z[:, 1152:1536]))
        parts.append(m[:, :360])
    m1 = jnp.concatenate(parts, axis=1)
    h1 = jnp.maximum(m1 + b1_ref[...], 0.0).astype(jnp.bfloat16)
    h1 = jnp.pad(h1, ((0, 0), (0, H1P - 1440)))
    w2 = w2_ref[...]
    parts = []
    for pr in range(2):
        z = jnp.dot(h1[:, 480 * pr:480 * pr + K2], w2,
                    preferred_element_type=jnp.float32)
        m = jnp.maximum(jnp.maximum(z[:, 0:256], z[:, 256:512]),
                        jnp.maximum(z[:, 512:768], z[:, 768:1024]))
        parts.append(m[:, :192])
    m2 = jnp.concatenate(parts, axis=1)
    h2 = jnp.maximum(m2 + b2_ref[...], 0.0).astype(jnp.bfloat16)
    z3 = jnp.dot(h2, w3_ref[...], preferred_element_type=jnp.float32)
    o_ref[...] = z3 + b3_ref[...]


def kernel(x, cw1, cb1, cw2, cb2, fw, fb):
    B = x.shape[0]
    TB = 1024 if B % 1024 == 0 else B

    xf = x.reshape(B, H1IN * W1IN)

    sel1 = jnp.asarray(_SEL1, jnp.bfloat16).reshape(-1, 32)
    w1s = cw1[:, :C1].astype(jnp.bfloat16)
    w1 = jax.lax.dot_general(sel1, w1s, (((1,), (0,)), ((), ())),
                             preferred_element_type=jnp.bfloat16)
    w1 = jnp.pad(w1.reshape(K1, 4, 360), ((0, 0), (0, 0), (0, 24)))
    w1 = w1.reshape(K1, N1)

    sel2 = jnp.asarray(_SEL2, jnp.bfloat16).reshape(-1, 256)
    w2s = cw2[:, :C2P].astype(jnp.bfloat16)
    w2 = jax.lax.dot_general(sel2, w2s, (((1,), (0,)), ((), ())),
                             preferred_element_type=jnp.bfloat16)
    w2 = jnp.pad(w2.reshape(K2, 4, 192), ((0, 0), (0, 0), (0, 64)))
    w2 = w2.reshape(K2, N2)

    b1 = jnp.broadcast_to(cb1[0:1, :C1], (P1H * P1W, C1)).reshape(1, 1440)

    b2 = jnp.broadcast_to(cb2[0:1, :C2P], (P2H * P2W, C2P)).reshape(1, G2)


    w3 = jnp.dot(jnp.asarray(_PERM3, jnp.float32), fw,
                 preferred_element_type=jnp.float32).astype(jnp.bfloat16)
    b3 = fb

    out = pl.pallas_call(
        _net_kernel,
        out_shape=jax.ShapeDtypeStruct((B, 128), jnp.float32),
        grid=(B // TB,),
        in_specs=[
            pl.BlockSpec((TB, H1IN * W1IN), lambda i: (i, 0)),
            pl.BlockSpec((K1, N1), lambda i: (0, 0)),
            pl.BlockSpec((1, 1440), lambda i: (0, 0)),
            pl.BlockSpec((K2, N2), lambda i: (0, 0)),
            pl.BlockSpec((1, G2), lambda i: (0, 0)),
            pl.BlockSpec((FPAD, 128), lambda i: (0, 0)),
            pl.BlockSpec((1, 128), lambda i: (0, 0)),
        ],
        out_specs=pl.BlockSpec((TB, 128), lambda i: (i, 0)),
        compiler_params=pltpu.CompilerParams(
            dimension_semantics=("parallel",),
            vmem_limit_bytes=64 * 1024 * 1024,
        ),
    )(xf, w1, b1, w2, b2, w3, b3)
    return out[:, :10]
